# Initial kernel scaffold; baseline (speedup 1.0000x reference)
import jax
import jax.numpy as jnp
from jax.experimental import pallas as pl


def kernel(cat_idx, subcat_idx, noise, category_dirs, subcategory_dirs):
    raise NotImplementedError("write your pallas kernel here")


# speedup vs baseline: 1.0280x; 1.0280x over previous
import functools

import jax
import jax.numpy as jnp
from jax import lax
from jax.experimental import pallas as pl
from jax.experimental.pallas import tpu as pltpu
from jax.experimental.pallas import tpu_sc as plsc

_B = 16384
_D = 64
_NCAT = 4
_NSUB = 12
_NC = 2
_NS = 16
_NW = _NC * _NS
_RPW = _B // _NW
_L = 16
_GROUPS = _RPW // _L


def _rsqrt_nr(x):
    i = plsc.bitcast(x, jnp.int32)
    i = jnp.int32(0x5F3759DF) - lax.shift_right_arithmetic(i, 1)
    y = plsc.bitcast(i, jnp.float32)
    for _ in range(3):
        y = y * (1.5 - 0.5 * x * y * y)
    return y


def _body(cat_hbm, sub_hbm, noise_hbm, catt_hbm, subt_hbm, out_hbm,
          ci_v, si_v, catt_v, subt_v, nz_v, out_v, emb_v):
    wid = lax.axis_index("s") * _NC + lax.axis_index("c")
    base = wid * _RPW

    pltpu.sync_copy(cat_hbm.at[pl.ds(base, _RPW)], ci_v)
    pltpu.sync_copy(sub_hbm.at[pl.ds(base, _RPW)], si_v)
    pltpu.sync_copy(catt_hbm, catt_v)
    pltpu.sync_copy(subt_hbm, subt_v)
    pltpu.sync_copy(noise_hbm.at[pl.ds(base * _D, _RPW * _D)], nz_v)

    lane64 = lax.iota(jnp.int32, _L) * _D

    def group(g, carry):
        r0 = g * _L
        catv = ci_v[pl.ds(r0, _L)]
        subv = si_v[pl.ds(r0, _L)]
        cbase = catv * _D
        sbase = subv * _D
        nbase = r0 * _D + lane64
        rad = 0.3 + (0.4 / 3.0) * (subv % 3).astype(jnp.float32)
        sq = jnp.zeros((_L,), jnp.float32)
        for d in range(_D):
            cvec = plsc.load_gather(catt_v, [cbase + d])
            svec = plsc.load_gather(subt_v, [sbase + d])
            nvec = plsc.load_gather(nz_v, [nbase + d])
            e = 0.4 * cvec + 0.5 * svec + 0.05 * nvec
            sq = sq + e * e
            emb_v[pl.ds(d * _L, _L)] = e
        s = jnp.maximum(sq, 1e-30)
        t = _rsqrt_nr(s)
        factor = rad / (s * t + 1e-6)
        for d in range(_D):
            e = emb_v[pl.ds(d * _L, _L)]
            plsc.store_scatter(out_v, [nbase + d], e * factor)
        return carry

    lax.fori_loop(0, _GROUPS, group, 0)

    pltpu.sync_copy(out_v, out_hbm.at[pl.ds(base * _D, _RPW * _D)])


def kernel(cat_idx, subcat_idx, noise, category_dirs, subcategory_dirs):
    mesh = plsc.VectorSubcoreMesh(core_axis_name="c", subcore_axis_name="s")
    run = pl.kernel(
        _body,
        mesh=mesh,
        out_type=jax.ShapeDtypeStruct((_B * _D,), jnp.float32),
        compiler_params=pltpu.CompilerParams(needs_layout_passes=False),
        scratch_types=[
            pltpu.VMEM((_RPW,), jnp.int32),
            pltpu.VMEM((_RPW,), jnp.int32),
            pltpu.VMEM((_NCAT * _D,), jnp.float32),
            pltpu.VMEM((_NSUB * _D,), jnp.float32),
            pltpu.VMEM((_RPW * _D,), jnp.float32),
            pltpu.VMEM((_RPW * _D,), jnp.float32),
            pltpu.VMEM((_L * _D,), jnp.float32),
        ],
    )
    out = run(cat_idx, subcat_idx, noise.reshape(-1),
              category_dirs.reshape(-1), subcategory_dirs.reshape(-1))
    return out.reshape(_B, _D)


# speedup vs baseline: 2.4029x; 2.3373x over previous
import jax
import jax.numpy as jnp
import numpy as np
from jax import lax
from jax.experimental import pallas as pl
from jax.experimental.pallas import tpu as pltpu
from jax.experimental.pallas import tpu_sc as plsc

_B = 16384
_D = 64
_NCAT = 4
_NSUB = 12
_NC = 2
_NS = 16
_NW = _NC * _NS
_RPW = _B // _NW
_L = 16
_GROUPS = _RPW // _L


def _body(cat_hbm, sub_hbm, noise_hbm, catt_hbm, subt_hbm, out_hbm,
          ci_v, si_v, catt_v, subt_v, combo_v, nz_v, out_v, rad_lut_v):
    wid = lax.axis_index("s") * _NC + lax.axis_index("c")
    base = wid * _RPW

    pltpu.sync_copy(cat_hbm.at[pl.ds(base, _RPW)], ci_v)
    pltpu.sync_copy(sub_hbm.at[pl.ds(base, _RPW)], si_v)
    pltpu.sync_copy(catt_hbm, catt_v)
    pltpu.sync_copy(subt_hbm, subt_v)
    pltpu.sync_copy(noise_hbm.at[pl.ds(base * _D, _RPW * _D)], nz_v)

    crows = [[8.0 * catt_v[pl.ds(c * _D + j * _L, _L)] for j in range(4)]
             for c in range(_NCAT)]
    for s in range(_NSUB):
        srow = [10.0 * subt_v[pl.ds(s * _D + j * _L, _L)] for j in range(4)]
        for c in range(_NCAT):
            for j in range(4):
                combo_v[pl.ds((c * _NSUB + s) * _D + j * _L, _L)] = (
                    crows[c][j] + srow[j])

    lane = lax.iota(jnp.int32, _L)
    rad_lut_v[...] = 0.3 + (0.4 / 3.0) * (lane % 3).astype(jnp.float32)
    lane_j = [lane + j * _L for j in range(4)]

    def group(g, carry):
        r0 = g * _L
        civ = ci_v[pl.ds(r0, _L)]
        siv = si_v[pl.ds(r0, _L)]
        coffv = (civ * _NSUB + siv) * _D
        radv = plsc.load_gather(rad_lut_v, [siv])
        for k0 in range(0, _L, 4):
            es = []
            svec = jnp.zeros((_L,), jnp.float32)
            for k in range(k0, k0 + 4):
                cb = jnp.broadcast_to(coffv[k], (_L,))
                nbase = (r0 + k) * _D
                e = [plsc.load_gather(combo_v, [cb + lane_j[j]])
                     + nz_v[pl.ds(nbase + j * _L, _L)] for j in range(4)]
                es.append(e)
                sq = ((e[0] * e[0] + e[1] * e[1])
                      + (e[2] * e[2] + e[3] * e[3]))
                tv = jnp.cumsum(sq)
                svec = jnp.where(lane == k,
                                 jnp.broadcast_to(tv[_L - 1], (_L,)), svec)
            s = jnp.maximum(svec, 1e-4)
            i = plsc.bitcast(s, jnp.int32)
            i = jnp.int32(0x5F3759DF) - lax.shift_right_arithmetic(i, 1)
            t = plsc.bitcast(i, jnp.float32)
            for _ in range(3):
                t = t * (1.5 - 0.5 * s * t * t)
            x = 2e-5 * t
            fvec = radv * (t * ((x * x - x) + 1.0))
            for i_k, k in enumerate(range(k0, k0 + 4)):
                fb = jnp.broadcast_to(fvec[k], (_L,))
                nbase = (r0 + k) * _D
                for j in range(4):
                    out_v[pl.ds(nbase + j * _L, _L)] = es[i_k][j] * fb
        return carry

    lax.fori_loop(0, _GROUPS, group, 0)

    pltpu.sync_copy(out_v, out_hbm.at[pl.ds(base * _D, _RPW * _D)])


def kernel(cat_idx, subcat_idx, noise, category_dirs, subcategory_dirs):
    mesh = plsc.VectorSubcoreMesh(core_axis_name="c", subcore_axis_name="s")
    run = pl.kernel(
        _body,
        mesh=mesh,
        out_type=jax.ShapeDtypeStruct((_B * _D,), jnp.float32),
        compiler_params=pltpu.CompilerParams(needs_layout_passes=False),
        scratch_types=[
            pltpu.VMEM((_RPW,), jnp.int32),
            pltpu.VMEM((_RPW,), jnp.int32),
            pltpu.VMEM((_NCAT * _D,), jnp.float32),
            pltpu.VMEM((_NSUB * _D,), jnp.float32),
            pltpu.VMEM((_NCAT * _NSUB * _D,), jnp.float32),
            pltpu.VMEM((_RPW * _D,), jnp.float32),
            pltpu.VMEM((_RPW * _D,), jnp.float32),
            pltpu.VMEM((_L,), jnp.float32),
        ],
    )
    out = run(cat_idx, subcat_idx, noise.reshape(-1),
              category_dirs.reshape(-1), subcategory_dirs.reshape(-1))
    return out.reshape(_B, _D)


# speedup vs baseline: 2.5293x; 1.0526x over previous
import jax
import jax.numpy as jnp
import numpy as np
from jax import lax
from jax.experimental import pallas as pl
from jax.experimental.pallas import tpu as pltpu
from jax.experimental.pallas import tpu_sc as plsc

_B = 16384
_D = 64
_NCAT = 4
_NSUB = 12
_NC = 2
_NS = 16
_NW = _NC * _NS
_RPW = _B // _NW
_L = 16
_GROUPS = _RPW // _L


def _body(cat_hbm, sub_hbm, noise_hbm, catt_hbm, subt_hbm, out_hbm,
          ci_v, si_v, catt_v, subt_v, combo_v, nz_v, out_v, rad_lut_v, sem):
    wid = lax.axis_index("s") * _NC + lax.axis_index("c")
    base = wid * _RPW

    cps = [
        pltpu.async_copy(cat_hbm.at[pl.ds(base, _RPW)], ci_v, sem),
        pltpu.async_copy(sub_hbm.at[pl.ds(base, _RPW)], si_v, sem),
        pltpu.async_copy(catt_hbm, catt_v, sem),
        pltpu.async_copy(subt_hbm, subt_v, sem),
        pltpu.async_copy(noise_hbm.at[pl.ds(base * _D, _RPW * _D)], nz_v,
                         sem),
    ]
    for cp in cps:
        cp.wait()

    crows = [[8.0 * catt_v[pl.ds(c * _D + j * _L, _L)] for j in range(4)]
             for c in range(_NCAT)]
    for s in range(_NSUB):
        srow = [10.0 * subt_v[pl.ds(s * _D + j * _L, _L)] for j in range(4)]
        for c in range(_NCAT):
            for j in range(4):
                combo_v[pl.ds((c * _NSUB + s) * _D + j * _L, _L)] = (
                    crows[c][j] + srow[j])

    lane = lax.iota(jnp.int32, _L)
    rad_lut_v[...] = 0.3 + (0.4 / 3.0) * (lane % 3).astype(jnp.float32)
    lane_j = [lane + j * _L for j in range(4)]

    def group(g, carry):
        r0 = g * _L
        civ = ci_v[pl.ds(r0, _L)]
        siv = si_v[pl.ds(r0, _L)]
        coffv = (civ * _NSUB + siv) * _D
        radv = plsc.load_gather(rad_lut_v, [siv])
        for k0 in range(0, _L, 4):
            es = []
            svec = jnp.zeros((_L,), jnp.float32)
            for k in range(k0, k0 + 4):
                cb = jnp.broadcast_to(coffv[k], (_L,))
                nbase = (r0 + k) * _D
                e = [plsc.load_gather(combo_v, [cb + lane_j[j]])
                     + nz_v[pl.ds(nbase + j * _L, _L)] for j in range(4)]
                es.append(e)
                sq = ((e[0] * e[0] + e[1] * e[1])
                      + (e[2] * e[2] + e[3] * e[3]))
                tv = jnp.cumsum(sq)
                svec = jnp.where(lane == k,
                                 jnp.broadcast_to(tv[_L - 1], (_L,)), svec)
            s = jnp.maximum(svec, 1e-4)
            i = plsc.bitcast(s, jnp.int32)
            i = jnp.int32(0x5F3759DF) - lax.shift_right_arithmetic(i, 1)
            t = plsc.bitcast(i, jnp.float32)
            for _ in range(3):
                t = t * (1.5 - 0.5 * s * t * t)
            x = 2e-5 * t
            fvec = radv * (t * (1.0 - x))
            for i_k, k in enumerate(range(k0, k0 + 4)):
                fb = jnp.broadcast_to(fvec[k], (_L,))
                nbase = (r0 + k) * _D
                for j in range(4):
                    out_v[pl.ds(nbase + j * _L, _L)] = es[i_k][j] * fb
        return carry

    half = _RPW * _D // 2
    lax.fori_loop(0, _GROUPS // 2, group, 0)
    ocp0 = pltpu.async_copy(out_v.at[pl.ds(0, half)],
                            out_hbm.at[pl.ds(base * _D, half)], sem)
    lax.fori_loop(_GROUPS // 2, _GROUPS, group, 0)
    ocp1 = pltpu.async_copy(out_v.at[pl.ds(half, half)],
                            out_hbm.at[pl.ds(base * _D + half, half)], sem)
    ocp0.wait()
    ocp1.wait()


def kernel(cat_idx, subcat_idx, noise, category_dirs, subcategory_dirs):
    mesh = plsc.VectorSubcoreMesh(core_axis_name="c", subcore_axis_name="s")
    run = pl.kernel(
        _body,
        mesh=mesh,
        out_type=jax.ShapeDtypeStruct((_B * _D,), jnp.float32),
        compiler_params=pltpu.CompilerParams(needs_layout_passes=False),
        scratch_types=[
            pltpu.VMEM((_RPW,), jnp.int32),
            pltpu.VMEM((_RPW,), jnp.int32),
            pltpu.VMEM((_NCAT * _D,), jnp.float32),
            pltpu.VMEM((_NSUB * _D,), jnp.float32),
            pltpu.VMEM((_NCAT * _NSUB * _D,), jnp.float32),
            pltpu.VMEM((_RPW * _D,), jnp.float32),
            pltpu.VMEM((_RPW * _D,), jnp.float32),
            pltpu.VMEM((_L,), jnp.float32),
            pltpu.SemaphoreType.DMA,
        ],
    )
    out = run(cat_idx, subcat_idx, noise.reshape(-1),
              category_dirs.reshape(-1), subcategory_dirs.reshape(-1))
    return out.reshape(_B, _D)


# speedup vs baseline: 3.3836x; 1.3377x over previous
import jax
import jax.numpy as jnp
from jax import lax
from jax.experimental import pallas as pl
from jax.experimental.pallas import tpu as pltpu
from jax.experimental.pallas import tpu_sc as plsc

_B = 16384
_D = 64
_NCAT = 4
_NSUB = 12
_NC = 2
_NS = 16
_NW = _NC * _NS
_RPW = _B // _NW
_L = 16
_CH = 128
_NCH = _RPW // _CH
_CG = _CH // _L


def _body(cat_hbm, sub_hbm, noise_hbm, catt_hbm, subt_hbm, out_hbm,
          ci_v, si_v, catt_v, subt_v, combo_v, rad_lut_v,
          nz0_v, nz1_v, ot0_v, ot1_v, sem, si0, si1, so0, so1):
    wid = lax.axis_index("s") * _NC + lax.axis_index("c")
    base = wid * _RPW
    nz = [nz0_v, nz1_v]
    ot = [ot0_v, ot1_v]
    sin = [si0, si1]
    son = [so0, so1]

    cps = [
        pltpu.async_copy(cat_hbm.at[pl.ds(base, _RPW)], ci_v, sem),
        pltpu.async_copy(sub_hbm.at[pl.ds(base, _RPW)], si_v, sem),
        pltpu.async_copy(catt_hbm, catt_v, sem),
        pltpu.async_copy(subt_hbm, subt_v, sem),
    ]
    in_cp = pltpu.async_copy(noise_hbm.at[pl.ds(base, _CH)], nz0_v, si0)
    for cp in cps:
        cp.wait()

    crows = [[8.0 * catt_v[c, pl.ds(j * _L, _L)] for j in range(4)]
             for c in range(_NCAT)]
    for s in range(_NSUB):
        srow = [10.0 * subt_v[s, pl.ds(j * _L, _L)] for j in range(4)]
        for c in range(_NCAT):
            for j in range(4):
                combo_v[pl.ds((c * _NSUB + s) * _D + j * _L, _L)] = (
                    crows[c][j] + srow[j])

    lane = lax.iota(jnp.int32, _L)
    rad_lut_v[...] = 0.3 + (0.4 / 3.0) * (lane % 3).astype(jnp.float32)
    lane_j = [lane + j * _L for j in range(4)]

    def make_group(nz_ref, ot_ref, ioff):
        def group(g, carry):
            r0 = g * _L
            civ = ci_v[pl.ds(ioff + r0, _L)]
            siv = si_v[pl.ds(ioff + r0, _L)]
            coffv = (civ * _NSUB + siv) * _D
            radv = plsc.load_gather(rad_lut_v, [siv])
            for k0 in range(0, _L, 4):
                es = []
                svec = jnp.zeros((_L,), jnp.float32)
                for k in range(k0, k0 + 4):
                    cb = jnp.broadcast_to(coffv[k], (_L,))
                    e = [plsc.load_gather(combo_v, [cb + lane_j[j]])
                         + nz_ref[r0 + k, pl.ds(j * _L, _L)]
                         for j in range(4)]
                    es.append(e)
                    sq = ((e[0] * e[0] + e[1] * e[1])
                          + (e[2] * e[2] + e[3] * e[3]))
                    tv = jnp.cumsum(sq)
                    svec = jnp.where(lane == k,
                                     jnp.broadcast_to(tv[_L - 1], (_L,)),
                                     svec)
                s = jnp.maximum(svec, 1e-4)
                i = plsc.bitcast(s, jnp.int32)
                i = jnp.int32(0x5F3759DF) - lax.shift_right_arithmetic(i, 1)
                t = plsc.bitcast(i, jnp.float32)
                for _ in range(3):
                    t = t * (1.5 - 0.5 * s * t * t)
                x = 2e-5 * t
                fvec = radv * (t * (1.0 - x))
                for i_k, k in enumerate(range(k0, k0 + 4)):
                    fb = jnp.broadcast_to(fvec[k], (_L,))
                    for j in range(4):
                        ot_ref[r0 + k, pl.ds(j * _L, _L)] = es[i_k][j] * fb
            return carry
        return group

    out_cp = [None, None]
    for c in range(_NCH):
        buf = c % 2
        if c + 1 < _NCH:
            nxt = pltpu.async_copy(
                noise_hbm.at[pl.ds(base + (c + 1) * _CH, _CH)],
                nz[1 - buf], sin[1 - buf])
        in_cp.wait()
        if out_cp[buf] is not None:
            out_cp[buf].wait()
        lax.fori_loop(0, _CG, make_group(nz[buf], ot[buf], c * _CH), 0)
        out_cp[buf] = pltpu.async_copy(
            ot[buf], out_hbm.at[pl.ds(base + c * _CH, _CH)], son[buf])
        if c + 1 < _NCH:
            in_cp = nxt
    out_cp[0].wait()
    out_cp[1].wait()


def kernel(cat_idx, subcat_idx, noise, category_dirs, subcategory_dirs):
    mesh = plsc.VectorSubcoreMesh(core_axis_name="c", subcore_axis_name="s")
    run = pl.kernel(
        _body,
        mesh=mesh,
        out_type=jax.ShapeDtypeStruct((_B, _D), jnp.float32),
        compiler_params=pltpu.CompilerParams(needs_layout_passes=False),
        scratch_types=[
            pltpu.VMEM((_RPW,), jnp.int32),
            pltpu.VMEM((_RPW,), jnp.int32),
            pltpu.VMEM((_NCAT, _D), jnp.float32),
            pltpu.VMEM((_NSUB, _D), jnp.float32),
            pltpu.VMEM((_NCAT * _NSUB * _D,), jnp.float32),
            pltpu.VMEM((_L,), jnp.float32),
            pltpu.VMEM((_CH, _D), jnp.float32),
            pltpu.VMEM((_CH, _D), jnp.float32),
            pltpu.VMEM((_CH, _D), jnp.float32),
            pltpu.VMEM((_CH, _D), jnp.float32),
            pltpu.SemaphoreType.DMA,
            pltpu.SemaphoreType.DMA,
            pltpu.SemaphoreType.DMA,
            pltpu.SemaphoreType.DMA,
            pltpu.SemaphoreType.DMA,
        ],
    )
    return run(cat_idx, subcat_idx, noise, category_dirs, subcategory_dirs)
